# baseline (speedup 1.0000x reference)
import functools

import jax
import jax.numpy as jnp
from jax import lax
from jax.experimental import pallas as pl
from jax.experimental.pallas import tpu as pltpu
from jax.experimental.pallas import tpu_sc as plsc

_N = 10000
_NSUB = 2500
_OUT = 256
_EPS = 1e-5

_QBLK = 400
_NBLK = _N // _QBLK
_KPAD = 2560
_PAD_COORD = 1e4

_NW = 32
_NPAD = 10240
_ROWS_PER_W = _NPAD // _NW
_CHUNK = 80
_NCHUNK = _ROWS_PER_W // _CHUNK
_LG = _OUT // 16


def _mlp_body(x_ref, w_ref, b_ref, gw_ref, gb_ref, gms_ref, o_ref):
    y = jnp.dot(x_ref[...], w_ref[...], preferred_element_type=jnp.float32)
    y = y + b_ref[...]
    mean = jnp.mean(y, axis=0, keepdims=True)
    c = y - gms_ref[...] * mean
    var = jnp.mean(c * c, axis=0, keepdims=True)
    z = gw_ref[...] * c / jnp.sqrt(var + _EPS) + gb_ref[...]
    o_ref[...] = jnp.maximum(z, 0.0)


def _top2_body(posq_ref, posk_ref, i1_ref, i2_ref, a1_ref, a2_ref):
    d = None
    for c in range(3):
        q = posq_ref[:, c:c + 1]
        k = posk_ref[c:c + 1, :]
        t = q - k
        d = t * t if d is None else d + t * t
    iota = lax.broadcasted_iota(jnp.int32, (_QBLK, _KPAD), 1)
    big = jnp.int32(2**30)
    m1 = jnp.min(d, axis=1, keepdims=True)
    i1 = jnp.min(jnp.where(d == m1, iota, big), axis=1, keepdims=True)
    dm = jnp.where(iota == i1, jnp.float32(jnp.inf), d)
    m2 = jnp.min(dm, axis=1, keepdims=True)
    i2 = jnp.min(jnp.where(dm == m2, iota, big), axis=1, keepdims=True)
    i1_ref[...] = i1
    i2_ref[...] = i2
    w1 = 1.0 / jnp.maximum(m1, 1e-16)
    w2 = 1.0 / jnp.maximum(m2, 1e-16)
    den = w1 + w2
    a1_ref[...] = jnp.broadcast_to(w1 / den, (_QBLK, 16))
    a2_ref[...] = jnp.broadcast_to(w2 / den, (_QBLK, 16))


@functools.lru_cache(maxsize=1)
def _make_sc_interp():
    @functools.partial(
        pl.kernel,
        mesh=plsc.VectorSubcoreMesh(core_axis_name="c", subcore_axis_name="s"),
        out_type=jax.ShapeDtypeStruct((_NPAD, _OUT), jnp.float32),
        scratch_types=[
            pltpu.VMEM((_CHUNK,), jnp.int32),
            pltpu.VMEM((_CHUNK,), jnp.int32),
            pltpu.VMEM((_CHUNK * 16,), jnp.float32),
            pltpu.VMEM((_CHUNK * 16,), jnp.float32),
            pltpu.VMEM((_CHUNK, _OUT), jnp.float32),
            pltpu.VMEM((_CHUNK, _OUT), jnp.float32),
            pltpu.SemaphoreType.DMA,
            pltpu.SemaphoreType.DMA,
        ],
    )
    def _sc_interp(table_hbm, idx1_hbm, idx2_hbm, a1_hbm, a2_hbm, out_hbm,
                   i1_v, i2_v, a1_v, a2_v, rows1_v, rows2_v, sem1, sem2):
        wid = lax.axis_index("s") * 2 + lax.axis_index("c")
        base = wid * _ROWS_PER_W
        for i in range(_NCHUNK):
            off = base + i * _CHUNK
            pltpu.sync_copy(idx1_hbm.at[pl.ds(off, _CHUNK)], i1_v)
            pltpu.sync_copy(idx2_hbm.at[pl.ds(off, _CHUNK)], i2_v)
            pltpu.sync_copy(a1_hbm.at[pl.ds(off * 16, _CHUNK * 16)], a1_v)
            pltpu.sync_copy(a2_hbm.at[pl.ds(off * 16, _CHUNK * 16)], a2_v)
            cp1 = pltpu.async_copy(table_hbm.at[i1_v], rows1_v, sem1)
            cp2 = pltpu.async_copy(table_hbm.at[i2_v], rows2_v, sem2)
            cp1.wait()
            cp2.wait()

            def _row(r, carry):
                a1s = a1_v[pl.ds(r * 16, 16)]
                a2s = a2_v[pl.ds(r * 16, 16)]
                for g in range(_LG):
                    f1 = rows1_v[r, pl.ds(g * 16, 16)]
                    f2 = rows2_v[r, pl.ds(g * 16, 16)]
                    rows1_v[r, pl.ds(g * 16, 16)] = a1s * f1 + a2s * f2
                return carry

            lax.fori_loop(0, _CHUNK, _row, 0)
            pltpu.sync_copy(rows1_v, out_hbm.at[pl.ds(off, _CHUNK)])

    return _sc_interp


def _gather_interp(table, idx1, idx2, a1p, a2p):
    return _make_sc_interp()(table, idx1, idx2, a1p, a2p)


def _mlp2_mm_body(x_ref, w_ref, b_ref, y_ref, s_ref):
    y = jnp.dot(x_ref[...], w_ref[...], preferred_element_type=jnp.float32)
    y = y + b_ref[...]
    y_ref[...] = y

    @pl.when(pl.program_id(0) == 0)
    def _init():
        s_ref[...] = jnp.zeros_like(s_ref)

    s_ref[0:1, :] += jnp.sum(y, axis=0, keepdims=True)
    s_ref[1:2, :] += jnp.sum(y * y, axis=0, keepdims=True)


def _norm_blend_body(y_ref, s_ref, gw_ref, gb_ref, gms_ref, interp_ref, o_ref):
    inv_n = jnp.float32(1.0 / _N)
    mean = s_ref[0:1, :] * inv_n
    ey2 = s_ref[1:2, :] * inv_n
    ms = gms_ref[...]
    var = ey2 - ms * (2.0 - ms) * mean * mean
    c = y_ref[...] - ms * mean
    z = gw_ref[...] * c / jnp.sqrt(var + _EPS) + gb_ref[...]
    o_ref[...] = jnp.maximum(z, 0.0) + interp_ref[...]


def kernel(x, x_sub, pos, pos_sub, batch, batch_sub,
           W1, b1, gw1, gb1, gms1, W2, b2, gw2, gb2, gms2):
    f32 = jnp.float32

    xs_t = pl.pallas_call(
        _mlp_body,
        out_shape=jax.ShapeDtypeStruct((_NSUB, _OUT), f32),
    )(x_sub, W1, b1.reshape(1, -1), gw1.reshape(1, -1),
      gb1.reshape(1, -1), gms1.reshape(1, -1))

    posk = jnp.full((8, _KPAD), _PAD_COORD, f32)
    posk = posk.at[:3, :_NSUB].set(pos_sub.T)
    i1, i2, a1, a2 = pl.pallas_call(
        _top2_body,
        grid=(_NBLK,),
        in_specs=[
            pl.BlockSpec((_QBLK, 3), lambda i: (i, 0)),
            pl.BlockSpec((8, _KPAD), lambda i: (0, 0)),
        ],
        out_specs=[
            pl.BlockSpec((_QBLK, 1), lambda i: (i, 0)),
            pl.BlockSpec((_QBLK, 1), lambda i: (i, 0)),
            pl.BlockSpec((_QBLK, 16), lambda i: (i, 0)),
            pl.BlockSpec((_QBLK, 16), lambda i: (i, 0)),
        ],
        out_shape=[
            jax.ShapeDtypeStruct((_N, 1), jnp.int32),
            jax.ShapeDtypeStruct((_N, 1), jnp.int32),
            jax.ShapeDtypeStruct((_N, 16), f32),
            jax.ShapeDtypeStruct((_N, 16), f32),
        ],
    )(pos, posk)

    idx1 = jnp.pad(i1.reshape(-1), (0, _NPAD - _N))
    idx2 = jnp.pad(i2.reshape(-1), (0, _NPAD - _N))
    a1p = jnp.pad(a1.reshape(-1), (0, (_NPAD - _N) * 16))
    a2p = jnp.pad(a2.reshape(-1), (0, (_NPAD - _N) * 16))
    interp = _gather_interp(xs_t, idx1, idx2, a1p, a2p)

    y, sums = pl.pallas_call(
        _mlp2_mm_body,
        grid=(_NBLK,),
        in_specs=[
            pl.BlockSpec((_QBLK, _OUT), lambda i: (i, 0)),
            pl.BlockSpec((_OUT, _OUT), lambda i: (0, 0)),
            pl.BlockSpec((1, _OUT), lambda i: (0, 0)),
        ],
        out_specs=[
            pl.BlockSpec((_QBLK, _OUT), lambda i: (i, 0)),
            pl.BlockSpec((8, _OUT), lambda i: (0, 0)),
        ],
        out_shape=[
            jax.ShapeDtypeStruct((_N, _OUT), f32),
            jax.ShapeDtypeStruct((8, _OUT), f32),
        ],
    )(x, W2, b2.reshape(1, -1))

    out = pl.pallas_call(
        _norm_blend_body,
        grid=(_NBLK,),
        in_specs=[
            pl.BlockSpec((_QBLK, _OUT), lambda i: (i, 0)),
            pl.BlockSpec((8, _OUT), lambda i: (0, 0)),
            pl.BlockSpec((1, _OUT), lambda i: (0, 0)),
            pl.BlockSpec((1, _OUT), lambda i: (0, 0)),
            pl.BlockSpec((1, _OUT), lambda i: (0, 0)),
            pl.BlockSpec((_QBLK, _OUT), lambda i: (i, 0)),
        ],
        out_specs=pl.BlockSpec((_QBLK, _OUT), lambda i: (i, 0)),
        out_shape=jax.ShapeDtypeStruct((_N, _OUT), f32),
    )(y, sums, gw2.reshape(1, -1), gb2.reshape(1, -1), gms2.reshape(1, -1),
      interp[:_N])
    return out


# speedup vs baseline: 1.1363x; 1.1363x over previous
import functools

import jax
import jax.numpy as jnp
from jax import lax
from jax.experimental import pallas as pl
from jax.experimental.pallas import tpu as pltpu
from jax.experimental.pallas import tpu_sc as plsc

_N = 10000
_NSUB = 2500
_OUT = 256
_EPS = 1e-5

_QBLK = 400
_NBLK = _N // _QBLK
_QBLK2 = 320
_KPAD = 2560
_PAD_COORD = 1e4

_NW = 32
_NPAD = 10240
_ROWS_PER_W = _NPAD // _NW
_CHUNK = 80
_NCHUNK = _ROWS_PER_W // _CHUNK
_LG = _OUT // 16


def _mlp_body(x_ref, w_ref, b_ref, gw_ref, gb_ref, gms_ref, o_ref):
    y = jnp.dot(x_ref[...], w_ref[...], preferred_element_type=jnp.float32)
    y = y + b_ref[...]
    mean = jnp.mean(y, axis=0, keepdims=True)
    c = y - gms_ref[...] * mean
    var = jnp.mean(c * c, axis=0, keepdims=True)
    z = gw_ref[...] * c / jnp.sqrt(var + _EPS) + gb_ref[...]
    o_ref[...] = jnp.maximum(z, 0.0)


def _top2_body(posq_ref, posk_ref, i1_ref, i2_ref, a1_ref, a2_ref):
    d = None
    for c in range(3):
        q = posq_ref[:, c:c + 1]
        k = posk_ref[c:c + 1, :]
        t = q - k
        d = t * t if d is None else d + t * t
    iota = lax.broadcasted_iota(jnp.int32, (_QBLK2, _KPAD), 1).astype(jnp.float32)
    big = jnp.float32(1e9)
    m1 = jnp.min(d, axis=1, keepdims=True)
    i1 = jnp.min(jnp.where(d == m1, iota, big), axis=1, keepdims=True)
    dm = jnp.where(iota == i1, jnp.float32(jnp.inf), d)
    m2 = jnp.min(dm, axis=1, keepdims=True)
    i2 = jnp.min(jnp.where(dm == m2, iota, big), axis=1, keepdims=True)
    i1_ref[...] = i1.astype(jnp.int32)
    i2_ref[...] = i2.astype(jnp.int32)
    w1 = 1.0 / jnp.maximum(m1, 1e-16)
    w2 = 1.0 / jnp.maximum(m2, 1e-16)
    den = w1 + w2
    a1_ref[...] = jnp.broadcast_to(w1 / den, (_QBLK2, 16))
    a2_ref[...] = jnp.broadcast_to(w2 / den, (_QBLK2, 16))


@functools.lru_cache(maxsize=1)
def _make_sc_interp():
    @functools.partial(
        pl.kernel,
        mesh=plsc.VectorSubcoreMesh(core_axis_name="c", subcore_axis_name="s"),
        out_type=jax.ShapeDtypeStruct((_NPAD, _OUT), jnp.float32),
        scratch_types=[
            pltpu.VMEM((2, _CHUNK), jnp.int32),
            pltpu.VMEM((2, _CHUNK), jnp.int32),
            pltpu.VMEM((2, _CHUNK * 16), jnp.float32),
            pltpu.VMEM((2, _CHUNK * 16), jnp.float32),
            pltpu.VMEM((2, _CHUNK, _OUT), jnp.float32),
            pltpu.VMEM((2, _CHUNK, _OUT), jnp.float32),
            pltpu.SemaphoreType.DMA,
            pltpu.SemaphoreType.DMA,
        ],
    )
    def _sc_interp(table_hbm, idx1_hbm, idx2_hbm, a1_hbm, a2_hbm, out_hbm,
                   i1_v, i2_v, a1_v, a2_v, rows1_v, rows2_v, sem1, sem2):
        wid = lax.axis_index("s") * 2 + lax.axis_index("c")
        base = wid * _ROWS_PER_W

        def _stage(c, b):
            off = base + c * _CHUNK
            pltpu.sync_copy(idx1_hbm.at[pl.ds(off, _CHUNK)], i1_v.at[b])
            pltpu.sync_copy(idx2_hbm.at[pl.ds(off, _CHUNK)], i2_v.at[b])
            pltpu.sync_copy(a1_hbm.at[pl.ds(off * 16, _CHUNK * 16)], a1_v.at[b])
            pltpu.sync_copy(a2_hbm.at[pl.ds(off * 16, _CHUNK * 16)], a2_v.at[b])
            c1 = pltpu.async_copy(table_hbm.at[i1_v.at[b]], rows1_v.at[b], sem1)
            c2 = pltpu.async_copy(table_hbm.at[i2_v.at[b]], rows2_v.at[b], sem2)
            return c1, c2

        pend = _stage(0, 0)
        for i in range(_NCHUNK):
            b = i % 2
            pend[0].wait()
            pend[1].wait()
            if i + 1 < _NCHUNK:
                pend = _stage(i + 1, (i + 1) % 2)

            def _row(r, carry):
                a1s = a1_v[b, pl.ds(r * 16, 16)]
                a2s = a2_v[b, pl.ds(r * 16, 16)]
                for g in range(_LG):
                    f1 = rows1_v[b, r, pl.ds(g * 16, 16)]
                    f2 = rows2_v[b, r, pl.ds(g * 16, 16)]
                    rows1_v[b, r, pl.ds(g * 16, 16)] = a1s * f1 + a2s * f2
                return carry

            lax.fori_loop(0, _CHUNK, _row, 0)
            off = base + i * _CHUNK
            pltpu.sync_copy(rows1_v.at[b], out_hbm.at[pl.ds(off, _CHUNK)])

    return _sc_interp


def _gather_interp(table, idx1, idx2, a1p, a2p):
    return _make_sc_interp()(table, idx1, idx2, a1p, a2p)


def _mlp2_mm_body(x_ref, w_ref, b_ref, y_ref, s_ref):
    y = jnp.dot(x_ref[...], w_ref[...], preferred_element_type=jnp.float32)
    y = y + b_ref[...]
    y_ref[...] = y

    @pl.when(pl.program_id(0) == 0)
    def _init():
        s_ref[...] = jnp.zeros_like(s_ref)

    s_ref[0:1, :] += jnp.sum(y, axis=0, keepdims=True)
    s_ref[1:2, :] += jnp.sum(y * y, axis=0, keepdims=True)


def _norm_blend_body(y_ref, s_ref, gw_ref, gb_ref, gms_ref, interp_ref, o_ref):
    inv_n = jnp.float32(1.0 / _N)
    mean = s_ref[0:1, :] * inv_n
    ey2 = s_ref[1:2, :] * inv_n
    ms = gms_ref[...]
    var = ey2 - ms * (2.0 - ms) * mean * mean
    c = y_ref[...] - ms * mean
    z = gw_ref[...] * c / jnp.sqrt(var + _EPS) + gb_ref[...]
    o_ref[...] = jnp.maximum(z, 0.0) + interp_ref[...]


def kernel(x, x_sub, pos, pos_sub, batch, batch_sub,
           W1, b1, gw1, gb1, gms1, W2, b2, gw2, gb2, gms2):
    f32 = jnp.float32

    xs_t = pl.pallas_call(
        _mlp_body,
        out_shape=jax.ShapeDtypeStruct((_NSUB, _OUT), f32),
    )(x_sub, W1, b1.reshape(1, -1), gw1.reshape(1, -1),
      gb1.reshape(1, -1), gms1.reshape(1, -1))

    posk = jnp.full((8, _KPAD), _PAD_COORD, f32)
    posk = posk.at[:3, :_NSUB].set(pos_sub.T)
    posq = jnp.pad(pos, ((0, _NPAD - _N), (0, 0)))
    i1, i2, a1, a2 = pl.pallas_call(
        _top2_body,
        grid=(_NPAD // _QBLK2,),
        in_specs=[
            pl.BlockSpec((_QBLK2, 3), lambda i: (i, 0)),
            pl.BlockSpec((8, _KPAD), lambda i: (0, 0)),
        ],
        out_specs=[
            pl.BlockSpec((_QBLK2, 1), lambda i: (i, 0)),
            pl.BlockSpec((_QBLK2, 1), lambda i: (i, 0)),
            pl.BlockSpec((_QBLK2, 16), lambda i: (i, 0)),
            pl.BlockSpec((_QBLK2, 16), lambda i: (i, 0)),
        ],
        out_shape=[
            jax.ShapeDtypeStruct((_NPAD, 1), jnp.int32),
            jax.ShapeDtypeStruct((_NPAD, 1), jnp.int32),
            jax.ShapeDtypeStruct((_NPAD, 16), f32),
            jax.ShapeDtypeStruct((_NPAD, 16), f32),
        ],
    )(posq, posk)

    interp = _gather_interp(xs_t, i1.reshape(-1), i2.reshape(-1),
                            a1.reshape(-1), a2.reshape(-1))

    y, sums = pl.pallas_call(
        _mlp2_mm_body,
        grid=(_NBLK,),
        in_specs=[
            pl.BlockSpec((_QBLK, _OUT), lambda i: (i, 0)),
            pl.BlockSpec((_OUT, _OUT), lambda i: (0, 0)),
            pl.BlockSpec((1, _OUT), lambda i: (0, 0)),
        ],
        out_specs=[
            pl.BlockSpec((_QBLK, _OUT), lambda i: (i, 0)),
            pl.BlockSpec((8, _OUT), lambda i: (0, 0)),
        ],
        out_shape=[
            jax.ShapeDtypeStruct((_N, _OUT), f32),
            jax.ShapeDtypeStruct((8, _OUT), f32),
        ],
    )(x, W2, b2.reshape(1, -1))

    out = pl.pallas_call(
        _norm_blend_body,
        grid=(_NBLK,),
        in_specs=[
            pl.BlockSpec((_QBLK, _OUT), lambda i: (i, 0)),
            pl.BlockSpec((8, _OUT), lambda i: (0, 0)),
            pl.BlockSpec((1, _OUT), lambda i: (0, 0)),
            pl.BlockSpec((1, _OUT), lambda i: (0, 0)),
            pl.BlockSpec((1, _OUT), lambda i: (0, 0)),
            pl.BlockSpec((_QBLK, _OUT), lambda i: (i, 0)),
        ],
        out_specs=pl.BlockSpec((_QBLK, _OUT), lambda i: (i, 0)),
        out_shape=jax.ShapeDtypeStruct((_N, _OUT), f32),
    )(y, sums, gw2.reshape(1, -1), gb2.reshape(1, -1), gms2.reshape(1, -1),
      interp[:_N])
    return out


# speedup vs baseline: 1.2048x; 1.0603x over previous
import functools

import jax
import jax.numpy as jnp
from jax import lax
from jax.experimental import pallas as pl
from jax.experimental.pallas import tpu as pltpu
from jax.experimental.pallas import tpu_sc as plsc

_N = 10000
_NSUB = 2500
_OUT = 256
_EPS = 1e-5

_QBLK = 400
_NBLK = _N // _QBLK
_QBLK2 = 320
_KPAD = 2560
_PAD_COORD = 1e4

_NW = 32
_NPAD = 10240
_ROWS_PER_W = _NPAD // _NW
_CHUNK = 64
_NCHUNK = _ROWS_PER_W // _CHUNK
_LG = _OUT // 16


def _mlp_body(x_ref, w_ref, b_ref, gw_ref, gb_ref, gms_ref, o_ref):
    y = jnp.dot(x_ref[...], w_ref[...], preferred_element_type=jnp.float32)
    y = y + b_ref[...]
    mean = jnp.mean(y, axis=0, keepdims=True)
    c = y - gms_ref[...] * mean
    var = jnp.mean(c * c, axis=0, keepdims=True)
    z = gw_ref[...] * c / jnp.sqrt(var + _EPS) + gb_ref[...]
    o_ref[...] = jnp.maximum(z, 0.0)


def _top2_body(posq_ref, posk_ref, i1_ref, i2_ref, a1_ref, a2_ref):
    d = None
    for c in range(3):
        q = posq_ref[:, c:c + 1]
        k = posk_ref[c:c + 1, :]
        t = q - k
        d = t * t if d is None else d + t * t
    iota = lax.broadcasted_iota(jnp.int32, (_QBLK2, _KPAD), 1).astype(jnp.float32)
    big = jnp.float32(1e9)
    m1 = jnp.min(d, axis=1, keepdims=True)
    i1 = jnp.min(jnp.where(d == m1, iota, big), axis=1, keepdims=True)
    dm = jnp.where(iota == i1, jnp.float32(jnp.inf), d)
    m2 = jnp.min(dm, axis=1, keepdims=True)
    i2 = jnp.min(jnp.where(dm == m2, iota, big), axis=1, keepdims=True)
    i1_ref[...] = i1.astype(jnp.int32)
    i2_ref[...] = i2.astype(jnp.int32)
    w1 = 1.0 / jnp.maximum(m1, 1e-16)
    w2 = 1.0 / jnp.maximum(m2, 1e-16)
    den = w1 + w2
    a1_ref[...] = jnp.broadcast_to(w1 / den, (_QBLK2, 16))
    a2_ref[...] = jnp.broadcast_to(w2 / den, (_QBLK2, 16))


@functools.lru_cache(maxsize=1)
def _make_sc_interp():
    @functools.partial(
        pl.kernel,
        mesh=plsc.VectorSubcoreMesh(core_axis_name="c", subcore_axis_name="s"),
        out_type=jax.ShapeDtypeStruct((_NPAD, _OUT), jnp.float32),
        scratch_types=[
            pltpu.VMEM((_ROWS_PER_W,), jnp.int32),
            pltpu.VMEM((_ROWS_PER_W,), jnp.int32),
            pltpu.VMEM((_ROWS_PER_W * 16,), jnp.float32),
            pltpu.VMEM((_ROWS_PER_W * 16,), jnp.float32),
            pltpu.VMEM((2, _CHUNK, _OUT), jnp.float32),
            pltpu.VMEM((2, _CHUNK, _OUT), jnp.float32),
            pltpu.VMEM((2, _CHUNK, _OUT), jnp.float32),
            pltpu.SemaphoreType.DMA,
            pltpu.SemaphoreType.DMA,
            pltpu.SemaphoreType.DMA,
            pltpu.SemaphoreType.DMA,
        ],
    )
    def _sc_interp(table_hbm, idx1_hbm, idx2_hbm, a1_hbm, a2_hbm, out_hbm,
                   i1_v, i2_v, a1_v, a2_v, rows1_v, rows2_v, out_v,
                   sem1, sem2, semw0, semw1):
        wid = lax.axis_index("s") * 2 + lax.axis_index("c")
        base = wid * _ROWS_PER_W

        pltpu.sync_copy(idx1_hbm.at[pl.ds(base, _ROWS_PER_W)], i1_v)
        pltpu.sync_copy(idx2_hbm.at[pl.ds(base, _ROWS_PER_W)], i2_v)
        pltpu.sync_copy(a1_hbm.at[pl.ds(base * 16, _ROWS_PER_W * 16)], a1_v)
        pltpu.sync_copy(a2_hbm.at[pl.ds(base * 16, _ROWS_PER_W * 16)], a2_v)

        def _gather(c):
            b = c % 2
            c1 = pltpu.async_copy(
                table_hbm.at[i1_v.at[pl.ds(c * _CHUNK, _CHUNK)]],
                rows1_v.at[b], sem1)
            c2 = pltpu.async_copy(
                table_hbm.at[i2_v.at[pl.ds(c * _CHUNK, _CHUNK)]],
                rows2_v.at[b], sem2)
            return c1, c2

        semw = (semw0, semw1)
        pend = [_gather(0)]
        if _NCHUNK > 1:
            pend.append(_gather(1))
        wpend = [None, None]
        for i in range(_NCHUNK):
            b = i % 2
            g1, g2 = pend[i]
            g1.wait()
            g2.wait()
            if wpend[b] is not None:
                wpend[b].wait()

            def _row(r, carry):
                a1s = a1_v[pl.ds(i * _CHUNK * 16 + r * 16, 16)]
                a2s = a2_v[pl.ds(i * _CHUNK * 16 + r * 16, 16)]
                for g in range(_LG):
                    f1 = rows1_v[b, r, pl.ds(g * 16, 16)]
                    f2 = rows2_v[b, r, pl.ds(g * 16, 16)]
                    out_v[b, r, pl.ds(g * 16, 16)] = a1s * f1 + a2s * f2
                return carry

            lax.fori_loop(0, _CHUNK, _row, 0)
            if i + 2 < _NCHUNK:
                pend.append(_gather(i + 2))
            off = base + i * _CHUNK
            wpend[b] = pltpu.async_copy(
                out_v.at[b], out_hbm.at[pl.ds(off, _CHUNK)], semw[b])
        for w in wpend:
            if w is not None:
                w.wait()

    return _sc_interp


def _gather_interp(table, idx1, idx2, a1p, a2p):
    return _make_sc_interp()(table, idx1, idx2, a1p, a2p)


def _mlp2_mm_body(x_ref, w_ref, b_ref, y_ref, s_ref):
    y = jnp.dot(x_ref[...], w_ref[...], preferred_element_type=jnp.float32)
    y = y + b_ref[...]
    y_ref[...] = y

    @pl.when(pl.program_id(0) == 0)
    def _init():
        s_ref[...] = jnp.zeros_like(s_ref)

    s_ref[0:1, :] += jnp.sum(y, axis=0, keepdims=True)
    s_ref[1:2, :] += jnp.sum(y * y, axis=0, keepdims=True)


def _norm_blend_body(y_ref, s_ref, gw_ref, gb_ref, gms_ref, interp_ref, o_ref):
    inv_n = jnp.float32(1.0 / _N)
    mean = s_ref[0:1, :] * inv_n
    ey2 = s_ref[1:2, :] * inv_n
    ms = gms_ref[...]
    var = ey2 - ms * (2.0 - ms) * mean * mean
    c = y_ref[...] - ms * mean
    z = gw_ref[...] * c / jnp.sqrt(var + _EPS) + gb_ref[...]
    o_ref[...] = jnp.maximum(z, 0.0) + interp_ref[...]


def kernel(x, x_sub, pos, pos_sub, batch, batch_sub,
           W1, b1, gw1, gb1, gms1, W2, b2, gw2, gb2, gms2):
    f32 = jnp.float32

    xs_t = pl.pallas_call(
        _mlp_body,
        out_shape=jax.ShapeDtypeStruct((_NSUB, _OUT), f32),
    )(x_sub, W1, b1.reshape(1, -1), gw1.reshape(1, -1),
      gb1.reshape(1, -1), gms1.reshape(1, -1))

    posk = jnp.full((8, _KPAD), _PAD_COORD, f32)
    posk = posk.at[:3, :_NSUB].set(pos_sub.T)
    posq = jnp.pad(pos, ((0, _NPAD - _N), (0, 0)))
    i1, i2, a1, a2 = pl.pallas_call(
        _top2_body,
        grid=(_NPAD // _QBLK2,),
        in_specs=[
            pl.BlockSpec((_QBLK2, 3), lambda i: (i, 0)),
            pl.BlockSpec((8, _KPAD), lambda i: (0, 0)),
        ],
        out_specs=[
            pl.BlockSpec((_QBLK2, 1), lambda i: (i, 0)),
            pl.BlockSpec((_QBLK2, 1), lambda i: (i, 0)),
            pl.BlockSpec((_QBLK2, 16), lambda i: (i, 0)),
            pl.BlockSpec((_QBLK2, 16), lambda i: (i, 0)),
        ],
        out_shape=[
            jax.ShapeDtypeStruct((_NPAD, 1), jnp.int32),
            jax.ShapeDtypeStruct((_NPAD, 1), jnp.int32),
            jax.ShapeDtypeStruct((_NPAD, 16), f32),
            jax.ShapeDtypeStruct((_NPAD, 16), f32),
        ],
    )(posq, posk)

    interp = _gather_interp(xs_t, i1.reshape(-1), i2.reshape(-1),
                            a1.reshape(-1), a2.reshape(-1))

    y, sums = pl.pallas_call(
        _mlp2_mm_body,
        grid=(_NBLK,),
        in_specs=[
            pl.BlockSpec((_QBLK, _OUT), lambda i: (i, 0)),
            pl.BlockSpec((_OUT, _OUT), lambda i: (0, 0)),
            pl.BlockSpec((1, _OUT), lambda i: (0, 0)),
        ],
        out_specs=[
            pl.BlockSpec((_QBLK, _OUT), lambda i: (i, 0)),
            pl.BlockSpec((8, _OUT), lambda i: (0, 0)),
        ],
        out_shape=[
            jax.ShapeDtypeStruct((_N, _OUT), f32),
            jax.ShapeDtypeStruct((8, _OUT), f32),
        ],
    )(x, W2, b2.reshape(1, -1))

    out = pl.pallas_call(
        _norm_blend_body,
        grid=(_NBLK,),
        in_specs=[
            pl.BlockSpec((_QBLK, _OUT), lambda i: (i, 0)),
            pl.BlockSpec((8, _OUT), lambda i: (0, 0)),
            pl.BlockSpec((1, _OUT), lambda i: (0, 0)),
            pl.BlockSpec((1, _OUT), lambda i: (0, 0)),
            pl.BlockSpec((1, _OUT), lambda i: (0, 0)),
            pl.BlockSpec((_QBLK, _OUT), lambda i: (i, 0)),
        ],
        out_specs=pl.BlockSpec((_QBLK, _OUT), lambda i: (i, 0)),
        out_shape=jax.ShapeDtypeStruct((_N, _OUT), f32),
    )(y, sums, gw2.reshape(1, -1), gb2.reshape(1, -1), gms2.reshape(1, -1),
      interp[:_N])
    return out


# speedup vs baseline: 1.3943x; 1.1573x over previous
import functools

import jax
import jax.numpy as jnp
from jax import lax
from jax.experimental import pallas as pl
from jax.experimental.pallas import tpu as pltpu
from jax.experimental.pallas import tpu_sc as plsc

_N = 10000
_NSUB = 2500
_OUT = 256
_EPS = 1e-5

_QBLK = 400
_NBLK = _N // _QBLK
_QBLK2 = 320
_KPAD = 2560
_PAD_COORD = 1e4

_NW = 32
_NPAD = 10240
_ROWS_PER_W = _NPAD // _NW
_CHUNK = 64
_NCHUNK = _ROWS_PER_W // _CHUNK
_LG = _OUT // 16


def _mlp_body(x_ref, w_ref, b_ref, gw_ref, gb_ref, gms_ref, o_ref):
    y = jnp.dot(x_ref[...], w_ref[...], preferred_element_type=jnp.float32)
    y = y + b_ref[...]
    mean = jnp.mean(y, axis=0, keepdims=True)
    c = y - gms_ref[...] * mean
    var = jnp.mean(c * c, axis=0, keepdims=True)
    z = gw_ref[...] * c / jnp.sqrt(var + _EPS) + gb_ref[...]
    o_ref[...] = jnp.maximum(z, 0.0)


def _top2_body(posq_ref, posk_ref, i1_ref, i2_ref, a1_ref):
    d = None
    for c in range(3):
        q = posq_ref[:, c:c + 1]
        k = posk_ref[c:c + 1, :]
        t = q - k
        d = t * t if d is None else d + t * t
    iota = lax.broadcasted_iota(jnp.int32, (_QBLK2, _KPAD), 1).astype(jnp.float32)
    big = jnp.float32(1e9)
    m1 = jnp.min(d, axis=1, keepdims=True)
    i1 = jnp.min(jnp.where(d == m1, iota, big), axis=1, keepdims=True)
    dm = jnp.where(iota == i1, jnp.float32(jnp.inf), d)
    m2 = jnp.min(dm, axis=1, keepdims=True)
    i2 = jnp.min(jnp.where(dm == m2, iota, big), axis=1, keepdims=True)
    nsub1 = jnp.float32(_NSUB - 1)
    i1_ref[...] = jnp.minimum(i1, nsub1).astype(jnp.int32)
    i2_ref[...] = jnp.minimum(i2, nsub1).astype(jnp.int32)
    w1 = 1.0 / jnp.maximum(m1, 1e-16)
    w2 = 1.0 / jnp.maximum(m2, 1e-16)
    a1_ref[...] = w1 / (w1 + w2)


@functools.lru_cache(maxsize=1)
def _make_sc_interp():
    @functools.partial(
        pl.kernel,
        mesh=plsc.VectorSubcoreMesh(core_axis_name="c", subcore_axis_name="s"),
        out_type=jax.ShapeDtypeStruct((_NPAD, _OUT), jnp.float32),
        scratch_types=[
            pltpu.VMEM((_ROWS_PER_W,), jnp.int32),
            pltpu.VMEM((_ROWS_PER_W,), jnp.int32),
            pltpu.VMEM((_ROWS_PER_W,), jnp.float32),
            pltpu.VMEM((2, _CHUNK, _OUT), jnp.float32),
            pltpu.VMEM((2, _CHUNK, _OUT), jnp.float32),
            pltpu.VMEM((2, _CHUNK, _OUT), jnp.float32),
            pltpu.SemaphoreType.DMA,
            pltpu.SemaphoreType.DMA,
            pltpu.SemaphoreType.DMA,
            pltpu.SemaphoreType.DMA,
        ],
    )
    def _sc_interp(table_hbm, idx1_hbm, idx2_hbm, a1_hbm, out_hbm,
                   i1_v, i2_v, a1_v, rows1_v, rows2_v, out_v,
                   sem1, sem2, semw0, semw1):
        wid = lax.axis_index("s") * 2 + lax.axis_index("c")
        base = wid * _ROWS_PER_W

        pltpu.sync_copy(idx1_hbm.at[pl.ds(base, _ROWS_PER_W)], i1_v)
        pltpu.sync_copy(idx2_hbm.at[pl.ds(base, _ROWS_PER_W)], i2_v)
        pltpu.sync_copy(a1_hbm.at[pl.ds(base, _ROWS_PER_W)], a1_v)

        def _gather(c):
            b = c % 2
            c1 = pltpu.async_copy(
                table_hbm.at[i1_v.at[pl.ds(c * _CHUNK, _CHUNK)]],
                rows1_v.at[b], sem1)
            c2 = pltpu.async_copy(
                table_hbm.at[i2_v.at[pl.ds(c * _CHUNK, _CHUNK)]],
                rows2_v.at[b], sem2)
            return c1, c2

        semw = (semw0, semw1)
        pend = [_gather(0)]
        if _NCHUNK > 1:
            pend.append(_gather(1))
        wpend = [None, None]
        for i in range(_NCHUNK):
            b = i % 2
            g1, g2 = pend[i]
            g1.wait()
            g2.wait()
            if wpend[b] is not None:
                wpend[b].wait()

            def _row(r, carry):
                grp = i * _CHUNK + (r & ~15)
                a1g = a1_v[pl.ds(grp, 16)]
                lane = jnp.full((16,), r & 15, jnp.int32)
                a1s = lax.gather(
                    a1g, lane[:, None],
                    lax.GatherDimensionNumbers(offset_dims=(),
                                               collapsed_slice_dims=(0,),
                                               start_index_map=(0,)),
                    (1,), mode=lax.GatherScatterMode.PROMISE_IN_BOUNDS)
                for g in range(_LG):
                    f1 = rows1_v[b, r, pl.ds(g * 16, 16)]
                    f2 = rows2_v[b, r, pl.ds(g * 16, 16)]
                    out_v[b, r, pl.ds(g * 16, 16)] = f2 + a1s * (f1 - f2)
                return carry

            lax.fori_loop(0, _CHUNK, _row, 0)
            if i + 2 < _NCHUNK:
                pend.append(_gather(i + 2))
            off = base + i * _CHUNK
            wpend[b] = pltpu.async_copy(
                out_v.at[b], out_hbm.at[pl.ds(off, _CHUNK)], semw[b])
        for w in wpend:
            if w is not None:
                w.wait()

    return _sc_interp


def _gather_interp(table, idx1, idx2, a1p):
    return _make_sc_interp()(table, idx1, idx2, a1p)


def _mlp2_mm_body(x_ref, w_ref, b_ref, y_ref, s_ref):
    y = jnp.dot(x_ref[...], w_ref[...], preferred_element_type=jnp.float32)
    y = y + b_ref[...]
    y_ref[...] = y

    @pl.when(pl.program_id(0) == 0)
    def _init():
        s_ref[...] = jnp.zeros_like(s_ref)

    s_ref[0:1, :] += jnp.sum(y, axis=0, keepdims=True)
    s_ref[1:2, :] += jnp.sum(y * y, axis=0, keepdims=True)


def _norm_blend_body(y_ref, s_ref, gw_ref, gb_ref, gms_ref, interp_ref, o_ref):
    inv_n = jnp.float32(1.0 / _N)
    mean = s_ref[0:1, :] * inv_n
    ey2 = s_ref[1:2, :] * inv_n
    ms = gms_ref[...]
    var = ey2 - ms * (2.0 - ms) * mean * mean
    c = y_ref[...] - ms * mean
    z = gw_ref[...] * c / jnp.sqrt(var + _EPS) + gb_ref[...]
    o_ref[...] = jnp.maximum(z, 0.0) + interp_ref[...]


def kernel(x, x_sub, pos, pos_sub, batch, batch_sub,
           W1, b1, gw1, gb1, gms1, W2, b2, gw2, gb2, gms2):
    f32 = jnp.float32

    xs_t = pl.pallas_call(
        _mlp_body,
        out_shape=jax.ShapeDtypeStruct((_NSUB, _OUT), f32),
    )(x_sub, W1, b1.reshape(1, -1), gw1.reshape(1, -1),
      gb1.reshape(1, -1), gms1.reshape(1, -1))

    posk = jnp.full((8, _KPAD), _PAD_COORD, f32)
    posk = posk.at[:3, :_NSUB].set(pos_sub.T)
    i1, i2, a1 = pl.pallas_call(
        _top2_body,
        grid=(_NPAD // _QBLK2,),
        in_specs=[
            pl.BlockSpec((_QBLK2, 3), lambda i: (i, 0)),
            pl.BlockSpec((8, _KPAD), lambda i: (0, 0)),
        ],
        out_specs=[
            pl.BlockSpec((_QBLK2, 1), lambda i: (i, 0)),
            pl.BlockSpec((_QBLK2, 1), lambda i: (i, 0)),
            pl.BlockSpec((_QBLK2, 1), lambda i: (i, 0)),
        ],
        out_shape=[
            jax.ShapeDtypeStruct((_NPAD, 1), jnp.int32),
            jax.ShapeDtypeStruct((_NPAD, 1), jnp.int32),
            jax.ShapeDtypeStruct((_NPAD, 1), f32),
        ],
    )(pos, posk)

    interp = _gather_interp(xs_t, i1.reshape(-1), i2.reshape(-1),
                            a1.reshape(-1))

    y, sums = pl.pallas_call(
        _mlp2_mm_body,
        grid=(_NBLK,),
        in_specs=[
            pl.BlockSpec((_QBLK, _OUT), lambda i: (i, 0)),
            pl.BlockSpec((_OUT, _OUT), lambda i: (0, 0)),
            pl.BlockSpec((1, _OUT), lambda i: (0, 0)),
        ],
        out_specs=[
            pl.BlockSpec((_QBLK, _OUT), lambda i: (i, 0)),
            pl.BlockSpec((8, _OUT), lambda i: (0, 0)),
        ],
        out_shape=[
            jax.ShapeDtypeStruct((_N, _OUT), f32),
            jax.ShapeDtypeStruct((8, _OUT), f32),
        ],
    )(x, W2, b2.reshape(1, -1))

    out = pl.pallas_call(
        _norm_blend_body,
        grid=(_NBLK,),
        in_specs=[
            pl.BlockSpec((_QBLK, _OUT), lambda i: (i, 0)),
            pl.BlockSpec((8, _OUT), lambda i: (0, 0)),
            pl.BlockSpec((1, _OUT), lambda i: (0, 0)),
            pl.BlockSpec((1, _OUT), lambda i: (0, 0)),
            pl.BlockSpec((1, _OUT), lambda i: (0, 0)),
            pl.BlockSpec((_QBLK, _OUT), lambda i: (i, 0)),
        ],
        out_specs=pl.BlockSpec((_QBLK, _OUT), lambda i: (i, 0)),
        out_shape=jax.ShapeDtypeStruct((_N, _OUT), f32),
    )(y, sums, gw2.reshape(1, -1), gb2.reshape(1, -1), gms2.reshape(1, -1),
      interp)
    return out


# speedup vs baseline: 1.4987x; 1.0748x over previous
import functools

import jax
import jax.numpy as jnp
from jax import lax
from jax.experimental import pallas as pl
from jax.experimental.pallas import tpu as pltpu
from jax.experimental.pallas import tpu_sc as plsc

_N = 10000
_NSUB = 2500
_OUT = 256
_EPS = 1e-5

_QBLK = 1000
_NBLK = _N // _QBLK
_QBLK2 = 320
_KPAD = 2560
_PAD_COORD = 1e4

_NW = 32
_NPAD = 10240
_ROWS_PER_W = _NPAD // _NW
_CHUNK = 64
_NCHUNK = _ROWS_PER_W // _CHUNK
_LG = _OUT // 16


def _mlp_body(x_ref, w_ref, b_ref, gw_ref, gb_ref, gms_ref, o_ref):
    y = jnp.dot(x_ref[...], w_ref[...], preferred_element_type=jnp.float32)
    y = y + b_ref[...]
    mean = jnp.mean(y, axis=0, keepdims=True)
    c = y - gms_ref[...] * mean
    var = jnp.mean(c * c, axis=0, keepdims=True)
    z = gw_ref[...] * c / jnp.sqrt(var + _EPS) + gb_ref[...]
    o_ref[...] = jnp.maximum(z, 0.0)


def _top2_body(posq_ref, posk_ref, i1_ref, i2_ref, a1_ref):
    d = None
    for c in range(3):
        q = posq_ref[:, c:c + 1]
        k = posk_ref[c:c + 1, :]
        t = q - k
        d = t * t if d is None else d + t * t
    iota = lax.broadcasted_iota(jnp.int32, (_QBLK2, _KPAD), 1).astype(jnp.float32)
    big = jnp.float32(1e9)
    m1 = jnp.min(d, axis=1, keepdims=True)
    i1 = jnp.min(jnp.where(d == m1, iota, big), axis=1, keepdims=True)
    dm = jnp.where(iota == i1, jnp.float32(jnp.inf), d)
    m2 = jnp.min(dm, axis=1, keepdims=True)
    i2 = jnp.min(jnp.where(dm == m2, iota, big), axis=1, keepdims=True)
    nsub1 = jnp.float32(_NSUB - 1)
    i1_ref[...] = jnp.minimum(i1, nsub1).astype(jnp.int32)
    i2_ref[...] = jnp.minimum(i2, nsub1).astype(jnp.int32)
    w1 = 1.0 / jnp.maximum(m1, 1e-16)
    w2 = 1.0 / jnp.maximum(m2, 1e-16)
    a1_ref[...] = w1 / (w1 + w2)


@functools.lru_cache(maxsize=1)
def _make_sc_interp():
    @functools.partial(
        pl.kernel,
        mesh=plsc.VectorSubcoreMesh(core_axis_name="c", subcore_axis_name="s"),
        out_type=jax.ShapeDtypeStruct((_NPAD, _OUT), jnp.float32),
        scratch_types=[
            pltpu.VMEM((_ROWS_PER_W,), jnp.int32),
            pltpu.VMEM((_ROWS_PER_W,), jnp.int32),
            pltpu.VMEM((_ROWS_PER_W,), jnp.float32),
            pltpu.VMEM((2, _CHUNK, _OUT), jnp.float32),
            pltpu.VMEM((2, _CHUNK, _OUT), jnp.float32),
            pltpu.VMEM((2, _CHUNK, _OUT), jnp.float32),
            pltpu.SemaphoreType.DMA,
            pltpu.SemaphoreType.DMA,
            pltpu.SemaphoreType.DMA,
            pltpu.SemaphoreType.DMA,
        ],
    )
    def _sc_interp(table_hbm, idx1_hbm, idx2_hbm, a1_hbm, out_hbm,
                   i1_v, i2_v, a1_v, rows1_v, rows2_v, out_v,
                   sem1, sem2, semw0, semw1):
        wid = lax.axis_index("s") * 2 + lax.axis_index("c")
        base = wid * _ROWS_PER_W

        pltpu.sync_copy(idx1_hbm.at[pl.ds(base, _ROWS_PER_W)], i1_v)
        pltpu.sync_copy(idx2_hbm.at[pl.ds(base, _ROWS_PER_W)], i2_v)
        pltpu.sync_copy(a1_hbm.at[pl.ds(base, _ROWS_PER_W)], a1_v)

        def _gather(c):
            b = c % 2
            c1 = pltpu.async_copy(
                table_hbm.at[i1_v.at[pl.ds(c * _CHUNK, _CHUNK)]],
                rows1_v.at[b], sem1)
            c2 = pltpu.async_copy(
                table_hbm.at[i2_v.at[pl.ds(c * _CHUNK, _CHUNK)]],
                rows2_v.at[b], sem2)
            return c1, c2

        semw = (semw0, semw1)
        pend = [_gather(0)]
        if _NCHUNK > 1:
            pend.append(_gather(1))
        wpend = [None, None]
        for i in range(_NCHUNK):
            b = i % 2
            g1, g2 = pend[i]
            g1.wait()
            g2.wait()
            if wpend[b] is not None:
                wpend[b].wait()

            def _row(r, carry):
                grp = i * _CHUNK + (r & ~15)
                a1g = a1_v[pl.ds(grp, 16)]
                lane = jnp.full((16,), r & 15, jnp.int32)
                a1s = lax.gather(
                    a1g, lane[:, None],
                    lax.GatherDimensionNumbers(offset_dims=(),
                                               collapsed_slice_dims=(0,),
                                               start_index_map=(0,)),
                    (1,), mode=lax.GatherScatterMode.PROMISE_IN_BOUNDS)
                for g in range(_LG):
                    f1 = rows1_v[b, r, pl.ds(g * 16, 16)]
                    f2 = rows2_v[b, r, pl.ds(g * 16, 16)]
                    out_v[b, r, pl.ds(g * 16, 16)] = f2 + a1s * (f1 - f2)
                return carry

            lax.fori_loop(0, _CHUNK, _row, 0)
            if i + 2 < _NCHUNK:
                pend.append(_gather(i + 2))
            off = base + i * _CHUNK
            wpend[b] = pltpu.async_copy(
                out_v.at[b], out_hbm.at[pl.ds(off, _CHUNK)], semw[b])
        for w in wpend:
            if w is not None:
                w.wait()

    return _sc_interp


def _gather_interp(table, idx1, idx2, a1p):
    return _make_sc_interp()(table, idx1, idx2, a1p)


def _mlp2_mm_body(x_ref, w_ref, b_ref, y_ref, s_ref):
    y = jnp.dot(x_ref[...], w_ref[...], preferred_element_type=jnp.float32)
    y = y + b_ref[...]
    y_ref[...] = y

    @pl.when(pl.program_id(0) == 0)
    def _init():
        s_ref[...] = jnp.zeros_like(s_ref)

    s_ref[0:1, :] += jnp.sum(y, axis=0, keepdims=True)
    s_ref[1:2, :] += jnp.sum(y * y, axis=0, keepdims=True)


def _norm_blend_body(y_ref, s_ref, gw_ref, gb_ref, gms_ref, interp_ref, o_ref):
    inv_n = jnp.float32(1.0 / _N)
    mean = s_ref[0:1, :] * inv_n
    ey2 = s_ref[1:2, :] * inv_n
    ms = gms_ref[...]
    var = ey2 - ms * (2.0 - ms) * mean * mean
    c = y_ref[...] - ms * mean
    z = gw_ref[...] * c / jnp.sqrt(var + _EPS) + gb_ref[...]
    o_ref[...] = jnp.maximum(z, 0.0) + interp_ref[...]


def kernel(x, x_sub, pos, pos_sub, batch, batch_sub,
           W1, b1, gw1, gb1, gms1, W2, b2, gw2, gb2, gms2):
    f32 = jnp.float32

    xs_t = pl.pallas_call(
        _mlp_body,
        out_shape=jax.ShapeDtypeStruct((_NSUB, _OUT), f32),
    )(x_sub, W1, b1.reshape(1, -1), gw1.reshape(1, -1),
      gb1.reshape(1, -1), gms1.reshape(1, -1))

    posk = jnp.full((8, _KPAD), _PAD_COORD, f32)
    posk = posk.at[:3, :_NSUB].set(pos_sub.T)
    i1, i2, a1 = pl.pallas_call(
        _top2_body,
        grid=(_NPAD // _QBLK2,),
        in_specs=[
            pl.BlockSpec((_QBLK2, 3), lambda i: (i, 0)),
            pl.BlockSpec((8, _KPAD), lambda i: (0, 0)),
        ],
        out_specs=[
            pl.BlockSpec((_QBLK2, 1), lambda i: (i, 0)),
            pl.BlockSpec((_QBLK2, 1), lambda i: (i, 0)),
            pl.BlockSpec((_QBLK2, 1), lambda i: (i, 0)),
        ],
        out_shape=[
            jax.ShapeDtypeStruct((_NPAD, 1), jnp.int32),
            jax.ShapeDtypeStruct((_NPAD, 1), jnp.int32),
            jax.ShapeDtypeStruct((_NPAD, 1), f32),
        ],
    )(pos, posk)

    interp = _gather_interp(xs_t, i1.reshape(-1), i2.reshape(-1),
                            a1.reshape(-1))

    y, sums = pl.pallas_call(
        _mlp2_mm_body,
        grid=(_NBLK,),
        in_specs=[
            pl.BlockSpec((_QBLK, _OUT), lambda i: (i, 0)),
            pl.BlockSpec((_OUT, _OUT), lambda i: (0, 0)),
            pl.BlockSpec((1, _OUT), lambda i: (0, 0)),
        ],
        out_specs=[
            pl.BlockSpec((_QBLK, _OUT), lambda i: (i, 0)),
            pl.BlockSpec((8, _OUT), lambda i: (0, 0)),
        ],
        out_shape=[
            jax.ShapeDtypeStruct((_N, _OUT), f32),
            jax.ShapeDtypeStruct((8, _OUT), f32),
        ],
    )(x, W2, b2.reshape(1, -1))

    out = pl.pallas_call(
        _norm_blend_body,
        grid=(_NBLK,),
        in_specs=[
            pl.BlockSpec((_QBLK, _OUT), lambda i: (i, 0)),
            pl.BlockSpec((8, _OUT), lambda i: (0, 0)),
            pl.BlockSpec((1, _OUT), lambda i: (0, 0)),
            pl.BlockSpec((1, _OUT), lambda i: (0, 0)),
            pl.BlockSpec((1, _OUT), lambda i: (0, 0)),
            pl.BlockSpec((_QBLK, _OUT), lambda i: (i, 0)),
        ],
        out_specs=pl.BlockSpec((_QBLK, _OUT), lambda i: (i, 0)),
        out_shape=jax.ShapeDtypeStruct((_N, _OUT), f32),
    )(y, sums, gw2.reshape(1, -1), gb2.reshape(1, -1), gms2.reshape(1, -1),
      interp)
    return out


# speedup vs baseline: 1.5254x; 1.0178x over previous
import functools

import jax
import jax.numpy as jnp
from jax import lax
from jax.experimental import pallas as pl
from jax.experimental.pallas import tpu as pltpu
from jax.experimental.pallas import tpu_sc as plsc

_N = 10000
_NSUB = 2500
_OUT = 256
_EPS = 1e-5

_QBLK = 1000
_NBLK = _N // _QBLK
_QBLK2 = 640
_KPAD = 2560
_PAD_COORD = 1e4

_NW = 32
_NPAD = 10240
_ROWS_PER_W = _NPAD // _NW
_CHUNK = 64
_NCHUNK = _ROWS_PER_W // _CHUNK
_LG = _OUT // 16


def _mlp_body(x_ref, w_ref, b_ref, gw_ref, gb_ref, gms_ref, o_ref):
    y = jnp.dot(x_ref[...], w_ref[...], preferred_element_type=jnp.float32)
    y = y + b_ref[...]
    mean = jnp.mean(y, axis=0, keepdims=True)
    c = y - gms_ref[...] * mean
    var = jnp.mean(c * c, axis=0, keepdims=True)
    z = gw_ref[...] * c / jnp.sqrt(var + _EPS) + gb_ref[...]
    o_ref[...] = jnp.maximum(z, 0.0)


def _top2_body(posq_ref, posk_ref, i1_ref, i2_ref, a1_ref):
    d = None
    for c in range(3):
        q = posq_ref[:, c:c + 1]
        k = posk_ref[c:c + 1, :]
        t = q - k
        d = t * t if d is None else d + t * t
    iota = lax.broadcasted_iota(jnp.int32, (_QBLK2, _KPAD), 1).astype(jnp.float32)
    big = jnp.float32(1e9)
    m1 = jnp.min(d, axis=1, keepdims=True)
    i1 = jnp.min(jnp.where(d == m1, iota, big), axis=1, keepdims=True)
    dm = jnp.where(iota == i1, jnp.float32(jnp.inf), d)
    m2 = jnp.min(dm, axis=1, keepdims=True)
    i2 = jnp.min(jnp.where(dm == m2, iota, big), axis=1, keepdims=True)
    nsub1 = jnp.float32(_NSUB - 1)
    i1_ref[...] = jnp.minimum(i1, nsub1).astype(jnp.int32)
    i2_ref[...] = jnp.minimum(i2, nsub1).astype(jnp.int32)
    w1 = 1.0 / jnp.maximum(m1, 1e-16)
    w2 = 1.0 / jnp.maximum(m2, 1e-16)
    a1_ref[...] = w1 / (w1 + w2)


@functools.lru_cache(maxsize=1)
def _make_sc_interp():
    @functools.partial(
        pl.kernel,
        mesh=plsc.VectorSubcoreMesh(core_axis_name="c", subcore_axis_name="s"),
        out_type=jax.ShapeDtypeStruct((_NPAD, _OUT), jnp.float32),
        scratch_types=[
            pltpu.VMEM((_ROWS_PER_W,), jnp.int32),
            pltpu.VMEM((_ROWS_PER_W,), jnp.int32),
            pltpu.VMEM((_ROWS_PER_W,), jnp.float32),
            pltpu.VMEM((2, _CHUNK, _OUT), jnp.float32),
            pltpu.VMEM((2, _CHUNK, _OUT), jnp.float32),
            pltpu.VMEM((2, _CHUNK, _OUT), jnp.float32),
            pltpu.SemaphoreType.DMA,
            pltpu.SemaphoreType.DMA,
            pltpu.SemaphoreType.DMA,
            pltpu.SemaphoreType.DMA,
        ],
    )
    def _sc_interp(table_hbm, idx1_hbm, idx2_hbm, a1_hbm, out_hbm,
                   i1_v, i2_v, a1_v, rows1_v, rows2_v, out_v,
                   sem1, sem2, semw0, semw1):
        wid = lax.axis_index("s") * 2 + lax.axis_index("c")
        base = wid * _ROWS_PER_W

        pltpu.sync_copy(idx1_hbm.at[pl.ds(base, _ROWS_PER_W)], i1_v)
        pltpu.sync_copy(idx2_hbm.at[pl.ds(base, _ROWS_PER_W)], i2_v)
        pltpu.sync_copy(a1_hbm.at[pl.ds(base, _ROWS_PER_W)], a1_v)

        def _gather(c):
            b = c % 2
            c1 = pltpu.async_copy(
                table_hbm.at[i1_v.at[pl.ds(c * _CHUNK, _CHUNK)]],
                rows1_v.at[b], sem1)
            c2 = pltpu.async_copy(
                table_hbm.at[i2_v.at[pl.ds(c * _CHUNK, _CHUNK)]],
                rows2_v.at[b], sem2)
            return c1, c2

        semw = (semw0, semw1)
        pend = [_gather(0)]
        if _NCHUNK > 1:
            pend.append(_gather(1))
        wpend = [None, None]
        for i in range(_NCHUNK):
            b = i % 2
            g1, g2 = pend[i]
            g1.wait()
            g2.wait()
            if wpend[b] is not None:
                wpend[b].wait()

            def _row(r, carry):
                grp = i * _CHUNK + (r & ~15)
                a1g = a1_v[pl.ds(grp, 16)]
                lane = jnp.full((16,), r & 15, jnp.int32)
                a1s = lax.gather(
                    a1g, lane[:, None],
                    lax.GatherDimensionNumbers(offset_dims=(),
                                               collapsed_slice_dims=(0,),
                                               start_index_map=(0,)),
                    (1,), mode=lax.GatherScatterMode.PROMISE_IN_BOUNDS)
                for g in range(_LG):
                    f1 = rows1_v[b, r, pl.ds(g * 16, 16)]
                    f2 = rows2_v[b, r, pl.ds(g * 16, 16)]
                    out_v[b, r, pl.ds(g * 16, 16)] = f2 + a1s * (f1 - f2)
                return carry

            lax.fori_loop(0, _CHUNK, _row, 0)
            if i + 2 < _NCHUNK:
                pend.append(_gather(i + 2))
            off = base + i * _CHUNK
            wpend[b] = pltpu.async_copy(
                out_v.at[b], out_hbm.at[pl.ds(off, _CHUNK)], semw[b])
        for w in wpend:
            if w is not None:
                w.wait()

    return _sc_interp


def _gather_interp(table, idx1, idx2, a1p):
    return _make_sc_interp()(table, idx1, idx2, a1p)


def _mlp2_mm_body(x_ref, w_ref, b_ref, y_ref, s_ref):
    y = jnp.dot(x_ref[...], w_ref[...], preferred_element_type=jnp.float32)
    y = y + b_ref[...]
    y_ref[...] = y

    @pl.when(pl.program_id(0) == 0)
    def _init():
        s_ref[...] = jnp.zeros_like(s_ref)

    s_ref[0:1, :] += jnp.sum(y, axis=0, keepdims=True)
    s_ref[1:2, :] += jnp.sum(y * y, axis=0, keepdims=True)


def _norm_blend_body(y_ref, s_ref, gw_ref, gb_ref, gms_ref, interp_ref, o_ref):
    inv_n = jnp.float32(1.0 / _N)
    mean = s_ref[0:1, :] * inv_n
    ey2 = s_ref[1:2, :] * inv_n
    ms = gms_ref[...]
    var = ey2 - ms * (2.0 - ms) * mean * mean
    c = y_ref[...] - ms * mean
    z = gw_ref[...] * c / jnp.sqrt(var + _EPS) + gb_ref[...]
    o_ref[...] = jnp.maximum(z, 0.0) + interp_ref[...]


def kernel(x, x_sub, pos, pos_sub, batch, batch_sub,
           W1, b1, gw1, gb1, gms1, W2, b2, gw2, gb2, gms2):
    f32 = jnp.float32

    xs_t = pl.pallas_call(
        _mlp_body,
        out_shape=jax.ShapeDtypeStruct((_NSUB, _OUT), f32),
    )(x_sub, W1, b1.reshape(1, -1), gw1.reshape(1, -1),
      gb1.reshape(1, -1), gms1.reshape(1, -1))

    posk = jnp.full((8, _KPAD), _PAD_COORD, f32)
    posk = posk.at[:3, :_NSUB].set(pos_sub.T)
    i1, i2, a1 = pl.pallas_call(
        _top2_body,
        grid=(_NPAD // _QBLK2,),
        in_specs=[
            pl.BlockSpec((_QBLK2, 3), lambda i: (i, 0)),
            pl.BlockSpec((8, _KPAD), lambda i: (0, 0)),
        ],
        out_specs=[
            pl.BlockSpec((_QBLK2, 1), lambda i: (i, 0)),
            pl.BlockSpec((_QBLK2, 1), lambda i: (i, 0)),
            pl.BlockSpec((_QBLK2, 1), lambda i: (i, 0)),
        ],
        out_shape=[
            jax.ShapeDtypeStruct((_NPAD, 1), jnp.int32),
            jax.ShapeDtypeStruct((_NPAD, 1), jnp.int32),
            jax.ShapeDtypeStruct((_NPAD, 1), f32),
        ],
    )(pos, posk)

    interp = _gather_interp(xs_t, i1.reshape(-1), i2.reshape(-1),
                            a1.reshape(-1))

    y, sums = pl.pallas_call(
        _mlp2_mm_body,
        grid=(_NBLK,),
        in_specs=[
            pl.BlockSpec((_QBLK, _OUT), lambda i: (i, 0)),
            pl.BlockSpec((_OUT, _OUT), lambda i: (0, 0)),
            pl.BlockSpec((1, _OUT), lambda i: (0, 0)),
        ],
        out_specs=[
            pl.BlockSpec((_QBLK, _OUT), lambda i: (i, 0)),
            pl.BlockSpec((8, _OUT), lambda i: (0, 0)),
        ],
        out_shape=[
            jax.ShapeDtypeStruct((_N, _OUT), f32),
            jax.ShapeDtypeStruct((8, _OUT), f32),
        ],
    )(x, W2, b2.reshape(1, -1))

    out = pl.pallas_call(
        _norm_blend_body,
        grid=(_NBLK,),
        in_specs=[
            pl.BlockSpec((_QBLK, _OUT), lambda i: (i, 0)),
            pl.BlockSpec((8, _OUT), lambda i: (0, 0)),
            pl.BlockSpec((1, _OUT), lambda i: (0, 0)),
            pl.BlockSpec((1, _OUT), lambda i: (0, 0)),
            pl.BlockSpec((1, _OUT), lambda i: (0, 0)),
            pl.BlockSpec((_QBLK, _OUT), lambda i: (i, 0)),
        ],
        out_specs=pl.BlockSpec((_QBLK, _OUT), lambda i: (i, 0)),
        out_shape=jax.ShapeDtypeStruct((_N, _OUT), f32),
    )(y, sums, gw2.reshape(1, -1), gb2.reshape(1, -1), gms2.reshape(1, -1),
      interp)
    return out


# speedup vs baseline: 1.6306x; 1.0690x over previous
import functools

import jax
import jax.numpy as jnp
from jax import lax
from jax.experimental import pallas as pl
from jax.experimental.pallas import tpu as pltpu
from jax.experimental.pallas import tpu_sc as plsc

_N = 10000
_NSUB = 2500
_OUT = 256
_EPS = 1e-5

_QBLK = 1000
_NBLK = _N // _QBLK
_QBLK2 = 640
_KPAD = 2560
_PAD_COORD = 1e4

_NW = 32
_NPAD = 10240
_ROWS_PER_W = _NPAD // _NW
_CHUNK = 64
_NCHUNK = _ROWS_PER_W // _CHUNK
_LG = _OUT // 16


def _mlp_body(x_ref, w_ref, b_ref, gw_ref, gb_ref, gms_ref, o_ref):
    y = jnp.dot(x_ref[...], w_ref[...], preferred_element_type=jnp.float32)
    y = y + b_ref[...]
    mean = jnp.mean(y, axis=0, keepdims=True)
    c = y - gms_ref[...] * mean
    var = jnp.mean(c * c, axis=0, keepdims=True)
    z = gw_ref[...] * c / jnp.sqrt(var + _EPS) + gb_ref[...]
    o_ref[...] = jnp.maximum(z, 0.0)


def _top2_body(posq_ref, posk_ref, i1_ref, i2_ref, a1_ref):
    d = None
    for c in range(3):
        q = posq_ref[:, c:c + 1]
        k = posk_ref[c:c + 1, :]
        t = q - k
        d = t * t if d is None else d + t * t
    iota = lax.broadcasted_iota(jnp.int32, (_QBLK2, _KPAD), 1).astype(jnp.float32)
    big = jnp.float32(1e9)
    m1 = jnp.min(d, axis=1, keepdims=True)
    i1 = jnp.min(jnp.where(d == m1, iota, big), axis=1, keepdims=True)
    dm = jnp.where(iota == i1, jnp.float32(jnp.inf), d)
    m2 = jnp.min(dm, axis=1, keepdims=True)
    i2 = jnp.min(jnp.where(dm == m2, iota, big), axis=1, keepdims=True)
    nsub1 = jnp.float32(_NSUB - 1)
    w1 = 1.0 / jnp.maximum(m1, 1e-16)
    w2 = 1.0 / jnp.maximum(m2, 1e-16)
    i1_ref[...] = jnp.minimum(i1, nsub1).astype(jnp.int32).T[None]
    i2_ref[...] = jnp.minimum(i2, nsub1).astype(jnp.int32).T[None]
    a1_ref[...] = (w1 / (w1 + w2)).T[None]


@functools.lru_cache(maxsize=1)
def _make_sc_interp():
    @functools.partial(
        pl.kernel,
        mesh=plsc.VectorSubcoreMesh(core_axis_name="c", subcore_axis_name="s"),
        out_type=jax.ShapeDtypeStruct((_NPAD, _OUT), jnp.float32),
        scratch_types=[
            pltpu.VMEM((_ROWS_PER_W,), jnp.int32),
            pltpu.VMEM((_ROWS_PER_W,), jnp.int32),
            pltpu.VMEM((_ROWS_PER_W,), jnp.float32),
            pltpu.VMEM((2, _CHUNK, _OUT), jnp.float32),
            pltpu.VMEM((2, _CHUNK, _OUT), jnp.float32),
            pltpu.VMEM((2, _CHUNK, _OUT), jnp.float32),
            pltpu.SemaphoreType.DMA,
            pltpu.SemaphoreType.DMA,
            pltpu.SemaphoreType.DMA,
            pltpu.SemaphoreType.DMA,
        ],
    )
    def _sc_interp(table_hbm, idx1_hbm, idx2_hbm, a1_hbm, out_hbm,
                   i1_v, i2_v, a1_v, rows1_v, rows2_v, out_v,
                   sem1, sem2, semw0, semw1):
        wid = lax.axis_index("s") * 2 + lax.axis_index("c")
        base = wid * _ROWS_PER_W

        pltpu.sync_copy(idx1_hbm.at[pl.ds(base, _ROWS_PER_W)], i1_v)
        pltpu.sync_copy(idx2_hbm.at[pl.ds(base, _ROWS_PER_W)], i2_v)
        acp = pltpu.async_copy(a1_hbm.at[pl.ds(base, _ROWS_PER_W)], a1_v, semw0)

        def _gather(c):
            b = c % 2
            c1 = pltpu.async_copy(
                table_hbm.at[i1_v.at[pl.ds(c * _CHUNK, _CHUNK)]],
                rows1_v.at[b], sem1)
            c2 = pltpu.async_copy(
                table_hbm.at[i2_v.at[pl.ds(c * _CHUNK, _CHUNK)]],
                rows2_v.at[b], sem2)
            return c1, c2

        semw = (semw0, semw1)
        pend = [_gather(0)]
        if _NCHUNK > 1:
            pend.append(_gather(1))
        acp.wait()
        wpend = [None, None]
        for i in range(_NCHUNK):
            b = i % 2
            g1, g2 = pend[i]
            g1.wait()
            g2.wait()
            if wpend[b] is not None:
                wpend[b].wait()

            def _row(r, carry):
                grp = i * _CHUNK + (r & ~15)
                a1g = a1_v[pl.ds(grp, 16)]
                lane = jnp.full((16,), r & 15, jnp.int32)
                a1s = lax.gather(
                    a1g, lane[:, None],
                    lax.GatherDimensionNumbers(offset_dims=(),
                                               collapsed_slice_dims=(0,),
                                               start_index_map=(0,)),
                    (1,), mode=lax.GatherScatterMode.PROMISE_IN_BOUNDS)
                for g in range(_LG):
                    f1 = rows1_v[b, r, pl.ds(g * 16, 16)]
                    f2 = rows2_v[b, r, pl.ds(g * 16, 16)]
                    out_v[b, r, pl.ds(g * 16, 16)] = f2 + a1s * (f1 - f2)
                return carry

            lax.fori_loop(0, _CHUNK, _row, 0)
            if i + 2 < _NCHUNK:
                pend.append(_gather(i + 2))
            off = base + i * _CHUNK
            wpend[b] = pltpu.async_copy(
                out_v.at[b], out_hbm.at[pl.ds(off, _CHUNK)], semw[b])
        for w in wpend:
            if w is not None:
                w.wait()

    return _sc_interp


def _gather_interp(table, idx1, idx2, a1p):
    return _make_sc_interp()(table, idx1, idx2, a1p)


def _mlp2_mm_body(x_ref, w_ref, b_ref, y_ref, s_ref):
    y = jnp.dot(x_ref[...], w_ref[...], preferred_element_type=jnp.float32)
    y = y + b_ref[...]
    y_ref[...] = y

    @pl.when(pl.program_id(0) == 0)
    def _init():
        s_ref[...] = jnp.zeros_like(s_ref)

    s_ref[0:1, :] += jnp.sum(y, axis=0, keepdims=True)
    s_ref[1:2, :] += jnp.sum(y * y, axis=0, keepdims=True)


def _norm_blend_body(y_ref, s_ref, gw_ref, gb_ref, gms_ref, interp_ref, o_ref):
    inv_n = jnp.float32(1.0 / _N)
    mean = s_ref[0:1, :] * inv_n
    ey2 = s_ref[1:2, :] * inv_n
    ms = gms_ref[...]
    var = ey2 - ms * (2.0 - ms) * mean * mean
    c = y_ref[...] - ms * mean
    z = gw_ref[...] * c / jnp.sqrt(var + _EPS) + gb_ref[...]
    o_ref[...] = jnp.maximum(z, 0.0) + interp_ref[...]


def kernel(x, x_sub, pos, pos_sub, batch, batch_sub,
           W1, b1, gw1, gb1, gms1, W2, b2, gw2, gb2, gms2):
    f32 = jnp.float32

    xs_t = pl.pallas_call(
        _mlp_body,
        out_shape=jax.ShapeDtypeStruct((_NSUB, _OUT), f32),
    )(x_sub, W1, b1.reshape(1, -1), gw1.reshape(1, -1),
      gb1.reshape(1, -1), gms1.reshape(1, -1))

    posk = jnp.full((8, _KPAD), _PAD_COORD, f32)
    posk = posk.at[:3, :_NSUB].set(pos_sub.T)
    i1, i2, a1 = pl.pallas_call(
        _top2_body,
        grid=(_NPAD // _QBLK2,),
        in_specs=[
            pl.BlockSpec((_QBLK2, 3), lambda i: (i, 0)),
            pl.BlockSpec((8, _KPAD), lambda i: (0, 0)),
        ],
        out_specs=[
            pl.BlockSpec((1, 1, _QBLK2), lambda i: (i, 0, 0)),
            pl.BlockSpec((1, 1, _QBLK2), lambda i: (i, 0, 0)),
            pl.BlockSpec((1, 1, _QBLK2), lambda i: (i, 0, 0)),
        ],
        out_shape=[
            jax.ShapeDtypeStruct((_NPAD // _QBLK2, 1, _QBLK2), jnp.int32),
            jax.ShapeDtypeStruct((_NPAD // _QBLK2, 1, _QBLK2), jnp.int32),
            jax.ShapeDtypeStruct((_NPAD // _QBLK2, 1, _QBLK2), f32),
        ],
    )(pos, posk)

    interp = _gather_interp(xs_t, i1.reshape(-1), i2.reshape(-1),
                            a1.reshape(-1))

    y, sums = pl.pallas_call(
        _mlp2_mm_body,
        grid=(_NBLK,),
        in_specs=[
            pl.BlockSpec((_QBLK, _OUT), lambda i: (i, 0)),
            pl.BlockSpec((_OUT, _OUT), lambda i: (0, 0)),
            pl.BlockSpec((1, _OUT), lambda i: (0, 0)),
        ],
        out_specs=[
            pl.BlockSpec((_QBLK, _OUT), lambda i: (i, 0)),
            pl.BlockSpec((8, _OUT), lambda i: (0, 0)),
        ],
        out_shape=[
            jax.ShapeDtypeStruct((_N, _OUT), f32),
            jax.ShapeDtypeStruct((8, _OUT), f32),
        ],
    )(x, W2, b2.reshape(1, -1))

    out = pl.pallas_call(
        _norm_blend_body,
        grid=(_NBLK,),
        in_specs=[
            pl.BlockSpec((_QBLK, _OUT), lambda i: (i, 0)),
            pl.BlockSpec((8, _OUT), lambda i: (0, 0)),
            pl.BlockSpec((1, _OUT), lambda i: (0, 0)),
            pl.BlockSpec((1, _OUT), lambda i: (0, 0)),
            pl.BlockSpec((1, _OUT), lambda i: (0, 0)),
            pl.BlockSpec((_QBLK, _OUT), lambda i: (i, 0)),
        ],
        out_specs=pl.BlockSpec((_QBLK, _OUT), lambda i: (i, 0)),
        out_shape=jax.ShapeDtypeStruct((_N, _OUT), f32),
    )(y, sums, gw2.reshape(1, -1), gb2.reshape(1, -1), gms2.reshape(1, -1),
      interp)
    return out


# speedup vs baseline: 1.6407x; 1.0062x over previous
import functools

import jax
import jax.numpy as jnp
from jax import lax
from jax.experimental import pallas as pl
from jax.experimental.pallas import tpu as pltpu
from jax.experimental.pallas import tpu_sc as plsc

_N = 10000
_NSUB = 2500
_IN = 512
_OUT = 256
_EPS = 1e-5

_QBLK = 1000
_NBLK = _N // _QBLK
_QBLK2 = 640
_KPAD = 2560
_PAD_COORD = 1e4

_NW = 32
_NPAD = 10240
_ROWS_PER_W = _NPAD // _NW
_CHUNK = 64
_NCHUNK = _ROWS_PER_W // _CHUNK
_LG = _OUT // 16


def _mlp_body(x_ref, w_ref, b_ref, gw_ref, gb_ref, gms_ref, o_ref):
    y = jnp.dot(x_ref[...], w_ref[...], preferred_element_type=jnp.float32)
    y = y + b_ref[...]
    mean = jnp.mean(y, axis=0, keepdims=True)
    c = y - gms_ref[...] * mean
    var = jnp.mean(c * c, axis=0, keepdims=True)
    z = gw_ref[...] * c / jnp.sqrt(var + _EPS) + gb_ref[...]
    o_ref[...] = jnp.maximum(z, 0.0)


def _top2_body(posq_ref, posk_ref, xs_ref, w1_ref, b1_ref, gw1_ref, gb1_ref,
               gms1_ref, i1_ref, i2_ref, a1_ref, xst_ref):
    @pl.when(pl.program_id(0) == 0)
    def _mlp1():
        _mlp_body(xs_ref, w1_ref, b1_ref, gw1_ref, gb1_ref, gms1_ref, xst_ref)
    d = None
    for c in range(3):
        q = posq_ref[:, c:c + 1]
        k = posk_ref[c:c + 1, :]
        t = q - k
        d = t * t if d is None else d + t * t
    iota = lax.broadcasted_iota(jnp.int32, (_QBLK2, _KPAD), 1).astype(jnp.float32)
    big = jnp.float32(1e9)
    m1 = jnp.min(d, axis=1, keepdims=True)
    i1 = jnp.min(jnp.where(d == m1, iota, big), axis=1, keepdims=True)
    dm = jnp.where(iota == i1, jnp.float32(jnp.inf), d)
    m2 = jnp.min(dm, axis=1, keepdims=True)
    i2 = jnp.min(jnp.where(dm == m2, iota, big), axis=1, keepdims=True)
    nsub1 = jnp.float32(_NSUB - 1)
    w1 = 1.0 / jnp.maximum(m1, 1e-16)
    w2 = 1.0 / jnp.maximum(m2, 1e-16)
    i1_ref[...] = jnp.minimum(i1, nsub1).astype(jnp.int32).T[None]
    i2_ref[...] = jnp.minimum(i2, nsub1).astype(jnp.int32).T[None]
    a1_ref[...] = (w1 / (w1 + w2)).T[None]


@functools.lru_cache(maxsize=1)
def _make_sc_interp():
    @functools.partial(
        pl.kernel,
        mesh=plsc.VectorSubcoreMesh(core_axis_name="c", subcore_axis_name="s"),
        out_type=jax.ShapeDtypeStruct((_NPAD, _OUT), jnp.float32),
        scratch_types=[
            pltpu.VMEM((_ROWS_PER_W,), jnp.int32),
            pltpu.VMEM((_ROWS_PER_W,), jnp.int32),
            pltpu.VMEM((_ROWS_PER_W,), jnp.float32),
            pltpu.VMEM((2, _CHUNK, _OUT), jnp.float32),
            pltpu.VMEM((2, _CHUNK, _OUT), jnp.float32),
            pltpu.VMEM((2, _CHUNK, _OUT), jnp.float32),
            pltpu.SemaphoreType.DMA,
            pltpu.SemaphoreType.DMA,
            pltpu.SemaphoreType.DMA,
            pltpu.SemaphoreType.DMA,
        ],
    )
    def _sc_interp(table_hbm, idx1_hbm, idx2_hbm, a1_hbm, out_hbm,
                   i1_v, i2_v, a1_v, rows1_v, rows2_v, out_v,
                   sem1, sem2, semw0, semw1):
        wid = lax.axis_index("s") * 2 + lax.axis_index("c")
        base = wid * _ROWS_PER_W

        pltpu.sync_copy(idx1_hbm.at[pl.ds(base, _ROWS_PER_W)], i1_v)
        pltpu.sync_copy(idx2_hbm.at[pl.ds(base, _ROWS_PER_W)], i2_v)
        acp = pltpu.async_copy(a1_hbm.at[pl.ds(base, _ROWS_PER_W)], a1_v, semw0)

        def _gather(c):
            b = c % 2
            c1 = pltpu.async_copy(
                table_hbm.at[i1_v.at[pl.ds(c * _CHUNK, _CHUNK)]],
                rows1_v.at[b], sem1)
            c2 = pltpu.async_copy(
                table_hbm.at[i2_v.at[pl.ds(c * _CHUNK, _CHUNK)]],
                rows2_v.at[b], sem2)
            return c1, c2

        semw = (semw0, semw1)
        pend = [_gather(0)]
        if _NCHUNK > 1:
            pend.append(_gather(1))
        acp.wait()
        wpend = [None, None]
        for i in range(_NCHUNK):
            b = i % 2
            g1, g2 = pend[i]
            g1.wait()
            g2.wait()
            if wpend[b] is not None:
                wpend[b].wait()

            def _row(r, carry):
                grp = i * _CHUNK + (r & ~15)
                a1g = a1_v[pl.ds(grp, 16)]
                lane = jnp.full((16,), r & 15, jnp.int32)
                a1s = lax.gather(
                    a1g, lane[:, None],
                    lax.GatherDimensionNumbers(offset_dims=(),
                                               collapsed_slice_dims=(0,),
                                               start_index_map=(0,)),
                    (1,), mode=lax.GatherScatterMode.PROMISE_IN_BOUNDS)
                for g in range(_LG):
                    f1 = rows1_v[b, r, pl.ds(g * 16, 16)]
                    f2 = rows2_v[b, r, pl.ds(g * 16, 16)]
                    out_v[b, r, pl.ds(g * 16, 16)] = f2 + a1s * (f1 - f2)
                return carry

            lax.fori_loop(0, _CHUNK, _row, 0)
            if i + 2 < _NCHUNK:
                pend.append(_gather(i + 2))
            off = base + i * _CHUNK
            wpend[b] = pltpu.async_copy(
                out_v.at[b], out_hbm.at[pl.ds(off, _CHUNK)], semw[b])
        for w in wpend:
            if w is not None:
                w.wait()

    return _sc_interp


def _gather_interp(table, idx1, idx2, a1p):
    return _make_sc_interp()(table, idx1, idx2, a1p)


def _mlp2_mm_body(x_ref, w_ref, b_ref, y_ref, s_ref):
    y = jnp.dot(x_ref[...], w_ref[...], preferred_element_type=jnp.float32)
    y = y + b_ref[...]
    y_ref[...] = y

    @pl.when(pl.program_id(0) == 0)
    def _init():
        s_ref[...] = jnp.zeros_like(s_ref)

    s_ref[0:1, :] += jnp.sum(y, axis=0, keepdims=True)
    s_ref[1:2, :] += jnp.sum(y * y, axis=0, keepdims=True)


def _norm_blend_body(y_ref, s_ref, gw_ref, gb_ref, gms_ref, interp_ref, o_ref):
    inv_n = jnp.float32(1.0 / _N)
    mean = s_ref[0:1, :] * inv_n
    ey2 = s_ref[1:2, :] * inv_n
    ms = gms_ref[...]
    var = ey2 - ms * (2.0 - ms) * mean * mean
    c = y_ref[...] - ms * mean
    z = gw_ref[...] * c / jnp.sqrt(var + _EPS) + gb_ref[...]
    o_ref[...] = jnp.maximum(z, 0.0) + interp_ref[...]


def kernel(x, x_sub, pos, pos_sub, batch, batch_sub,
           W1, b1, gw1, gb1, gms1, W2, b2, gw2, gb2, gms2):
    f32 = jnp.float32

    posk = jnp.full((8, _KPAD), _PAD_COORD, f32)
    posk = posk.at[:3, :_NSUB].set(pos_sub.T)
    i1, i2, a1, xs_t = pl.pallas_call(
        _top2_body,
        grid=(_NPAD // _QBLK2,),
        in_specs=[
            pl.BlockSpec((_QBLK2, 3), lambda i: (i, 0)),
            pl.BlockSpec((8, _KPAD), lambda i: (0, 0)),
            pl.BlockSpec((_NSUB, _IN), lambda i: (0, 0)),
            pl.BlockSpec((_IN, _OUT), lambda i: (0, 0)),
            pl.BlockSpec((1, _OUT), lambda i: (0, 0)),
            pl.BlockSpec((1, _OUT), lambda i: (0, 0)),
            pl.BlockSpec((1, _OUT), lambda i: (0, 0)),
            pl.BlockSpec((1, _OUT), lambda i: (0, 0)),
        ],
        out_specs=[
            pl.BlockSpec((1, 1, _QBLK2), lambda i: (i, 0, 0)),
            pl.BlockSpec((1, 1, _QBLK2), lambda i: (i, 0, 0)),
            pl.BlockSpec((1, 1, _QBLK2), lambda i: (i, 0, 0)),
            pl.BlockSpec((_NSUB, _OUT), lambda i: (0, 0)),
        ],
        out_shape=[
            jax.ShapeDtypeStruct((_NPAD // _QBLK2, 1, _QBLK2), jnp.int32),
            jax.ShapeDtypeStruct((_NPAD // _QBLK2, 1, _QBLK2), jnp.int32),
            jax.ShapeDtypeStruct((_NPAD // _QBLK2, 1, _QBLK2), f32),
            jax.ShapeDtypeStruct((_NSUB, _OUT), f32),
        ],
    )(pos, posk, x_sub, W1, b1.reshape(1, -1), gw1.reshape(1, -1),
      gb1.reshape(1, -1), gms1.reshape(1, -1))

    interp = _gather_interp(xs_t, i1.reshape(-1), i2.reshape(-1),
                            a1.reshape(-1))

    y, sums = pl.pallas_call(
        _mlp2_mm_body,
        grid=(_NBLK,),
        in_specs=[
            pl.BlockSpec((_QBLK, _OUT), lambda i: (i, 0)),
            pl.BlockSpec((_OUT, _OUT), lambda i: (0, 0)),
            pl.BlockSpec((1, _OUT), lambda i: (0, 0)),
        ],
        out_specs=[
            pl.BlockSpec((_QBLK, _OUT), lambda i: (i, 0)),
            pl.BlockSpec((8, _OUT), lambda i: (0, 0)),
        ],
        out_shape=[
            jax.ShapeDtypeStruct((_N, _OUT), f32),
            jax.ShapeDtypeStruct((8, _OUT), f32),
        ],
    )(x, W2, b2.reshape(1, -1))

    out = pl.pallas_call(
        _norm_blend_body,
        grid=(_NBLK,),
        in_specs=[
            pl.BlockSpec((_QBLK, _OUT), lambda i: (i, 0)),
            pl.BlockSpec((8, _OUT), lambda i: (0, 0)),
            pl.BlockSpec((1, _OUT), lambda i: (0, 0)),
            pl.BlockSpec((1, _OUT), lambda i: (0, 0)),
            pl.BlockSpec((1, _OUT), lambda i: (0, 0)),
            pl.BlockSpec((_QBLK, _OUT), lambda i: (i, 0)),
        ],
        out_specs=pl.BlockSpec((_QBLK, _OUT), lambda i: (i, 0)),
        out_shape=jax.ShapeDtypeStruct((_N, _OUT), f32),
    )(y, sums, gw2.reshape(1, -1), gb2.reshape(1, -1), gms2.reshape(1, -1),
      interp)
    return out
